# baseline (speedup 1.0000x reference)
import functools

import jax
import jax.numpy as jnp
from jax import lax
from jax.experimental import pallas as pl
from jax.experimental.pallas import tpu as pltpu
from jax.experimental.pallas import tpu_sc as plsc

N = 2048
D = 1024
F = 2048
E = 8
K = 2
NF = N * K
CAP = 1024
TM = 128
CB = CAP // TM
NC, NS, L = 2, 16, 16
NW = NC * NS
SPT = NF // NW
VPT = SPT // L
NCHUNK = 4
CH = SPT // NCHUNK


def _dispatch_body(x_hbm, ei_hbm, xbuf_hbm, inv_hbm,
                   ei_v, tok_idx, inv_idx, shift, rows0, rows1, rows2,
                   gsem, s0sem, s1sem, s2sem):
    cid = lax.axis_index("c")
    sid = lax.axis_index("s")
    wid = sid * NC + cid
    base = wid * SPT

    pltpu.sync_copy(ei_hbm, ei_v)
    shift[pl.ds(0, L)] = jnp.zeros((L,), jnp.int32)

    iota = lax.iota(jnp.int32, L)
    for j in range(VPT):
        c_idx = j // (VPT // NCHUNK)
        off = (j % (VPT // NCHUNK)) * L
        tok_idx[c_idx, pl.ds(off, L)] = (
            jnp.full((L,), base + j * L, jnp.int32) + iota) >> 1

    bufs = (rows0, rows1, rows2)
    gh = [pltpu.async_copy(x_hbm.at[tok_idx.at[c]], bufs[c], gsem)
          for c in range(3)]

    def prefix_incl(x):
        for k in (1, 2, 4, 8):
            shift[pl.ds(L, L)] = x
            x = x + shift[pl.ds(L - k, L)]
        return x

    def splat_last(x):
        return jnp.full((L,), x[L - 1], jnp.int32)

    def hist_step(j, accs):
        v = ei_v[pl.ds(j * L, L)]
        return tuple(
            acc + jnp.where(v == e, 1, 0) for e, acc in enumerate(accs))

    zeros = tuple(jnp.zeros((L,), jnp.int32) for _ in range(E))
    accs = lax.fori_loop(0, wid * VPT, hist_step, zeros)
    cnts = [splat_last(prefix_incl(a)) for a in accs]

    for j in range(VPT):
        v = ei_v[pl.ds(base + j * L, L)]
        inv_j = jnp.zeros((L,), jnp.int32)
        for e in range(E):
            m = v == e
            csum = prefix_incl(jnp.where(m, 1, 0))
            inv_j = jnp.where(m, e * CAP + cnts[e] + csum - 1, inv_j)
            cnts[e] = cnts[e] + splat_last(csum)
        c_idx = j // (VPT // NCHUNK)
        off = (j % (VPT // NCHUNK)) * L
        inv_idx[c_idx, pl.ds(off, L)] = inv_j

    pltpu.sync_copy(inv_idx, inv_hbm.at[wid])

    ssems = (s0sem, s1sem, s2sem)
    sh = []
    for c in range(3):
        gh[c].wait()
        sh.append(pltpu.async_copy(bufs[c], xbuf_hbm.at[inv_idx.at[c]],
                                   ssems[c]))
    sh[0].wait()
    g3 = pltpu.async_copy(x_hbm.at[tok_idx.at[3]], bufs[0], gsem)
    g3.wait()
    sh.append(pltpu.async_copy(bufs[0], xbuf_hbm.at[inv_idx.at[3]], s0sem))
    sh[1].wait()
    sh[2].wait()
    sh[3].wait()


def _gemm_body(counts_ref, x_ref, w1_ref, w2_ref, o_ref, w1b, w2b):
    e = pl.program_id(0)
    c = pl.program_id(1)

    @pl.when(c == 0)
    def _cast():
        w1b[...] = w1_ref[0].astype(jnp.bfloat16)
        w2b[...] = w2_ref[0].astype(jnp.bfloat16)

    @pl.when(c * TM < counts_ref[e])
    def _():
        x = x_ref[...].astype(jnp.bfloat16)
        h = jax.nn.relu(
            lax.dot_general(x, w1b[...], (((1,), (0,)), ((), ())),
                            preferred_element_type=jnp.float32)
        ).astype(jnp.bfloat16)
        o_ref[...] = lax.dot_general(h, w2b[...], (((1,), (0,)), ((), ())),
                                     preferred_element_type=jnp.float32)


def _combine_body(ybuf_hbm, inv_hbm, ew_hbm, out_hbm,
                  inv_idx, ew_v, rows0, rows1, orows, gsem):
    cid = lax.axis_index("c")
    sid = lax.axis_index("s")
    wid = sid * NC + cid

    pltpu.sync_copy(inv_hbm.at[wid], inv_idx)
    pltpu.sync_copy(ew_hbm.at[pl.ds(wid * SPT, SPT)], ew_v.at[pl.ds(0, SPT)])

    bufs = (rows0, rows1)
    g = pltpu.async_copy(ybuf_hbm.at[inv_idx.at[0]], bufs[0], gsem)
    for c in range(NCHUNK):
        g.wait()
        if c + 1 < NCHUNK:
            g_next = pltpu.async_copy(
                ybuf_hbm.at[inv_idx.at[c + 1]], bufs[(c + 1) % 2], gsem)
        rows = bufs[c % 2]

        def tbody(t, _):
            w0 = jnp.full((L,), ew_v[pl.ds(c * CH + 2 * t, L)][0], jnp.float32)
            w1 = jnp.full((L,), ew_v[pl.ds(c * CH + 2 * t + 1, L)][0],
                          jnp.float32)
            for jj in range(D // L):
                s = pl.ds(jj * L, L)
                orows[t, s] = w0 * rows[2 * t, s] + w1 * rows[2 * t + 1, s]
            return 0

        lax.fori_loop(0, CH // K, tbody, 0)
        pltpu.sync_copy(
            orows, out_hbm.at[pl.ds(wid * (SPT // K) + c * (CH // K), CH // K)])
        if c + 1 < NCHUNK:
            g = g_next


@functools.lru_cache(maxsize=None)
def _sc_kernels():
    mesh = plsc.VectorSubcoreMesh(
        core_axis_name="c", subcore_axis_name="s", num_cores=NC, num_subcores=NS)
    dispatch = pl.kernel(
        _dispatch_body,
        out_type=[
            jax.ShapeDtypeStruct((E * CAP, D), jnp.float32),
            jax.ShapeDtypeStruct((NW, NCHUNK, CH), jnp.int32),
        ],
        mesh=mesh,
        scratch_types=[
            pltpu.VMEM((NF,), jnp.int32),
            pltpu.VMEM((NCHUNK, CH), jnp.int32),
            pltpu.VMEM((NCHUNK, CH), jnp.int32),
            pltpu.VMEM((2 * L,), jnp.int32),
            pltpu.VMEM((CH, D), jnp.float32),
            pltpu.VMEM((CH, D), jnp.float32),
            pltpu.VMEM((CH, D), jnp.float32),
            pltpu.SemaphoreType.DMA,
            pltpu.SemaphoreType.DMA,
            pltpu.SemaphoreType.DMA,
            pltpu.SemaphoreType.DMA,
        ],
    )
    combine = pl.kernel(
        _combine_body,
        out_type=jax.ShapeDtypeStruct((N, D), jnp.float32),
        mesh=mesh,
        scratch_types=[
            pltpu.VMEM((NCHUNK, CH), jnp.int32),
            pltpu.VMEM((SPT + L,), jnp.float32),
            pltpu.VMEM((CH, D), jnp.float32),
            pltpu.VMEM((CH, D), jnp.float32),
            pltpu.VMEM((CH // K, D), jnp.float32),
            pltpu.SemaphoreType.DMA,
        ],
    )
    return dispatch, combine

_gemm = pl.pallas_call(
    _gemm_body,
    grid_spec=pltpu.PrefetchScalarGridSpec(
        num_scalar_prefetch=1,
        grid=(E, CB),
        in_specs=[
            pl.BlockSpec(
                (TM, D),
                lambda e, c, cnt: (
                    e * CB + jnp.minimum(
                        c, jnp.maximum((cnt[e] + TM - 1) // TM - 1, 0)), 0)),
            pl.BlockSpec((1, D, F), lambda e, c, cnt: (e, 0, 0)),
            pl.BlockSpec((1, F, D), lambda e, c, cnt: (e, 0, 0)),
        ],
        out_specs=pl.BlockSpec(
            (TM, D),
            lambda e, c, cnt: (
                jnp.where(c * TM < cnt[e], e * CB + c, E * CB), 0)),
        scratch_shapes=[
            pltpu.VMEM((D, F), jnp.bfloat16),
            pltpu.VMEM((F, D), jnp.bfloat16),
        ],
    ),
    out_shape=jax.ShapeDtypeStruct(((E * CB + 1) * TM, D), jnp.float32),
)


def kernel(x, expert_weights, expert_indices, batch_size_per_expert, W1, W2):
    ei_flat = expert_indices.astype(jnp.int32).reshape(-1)
    ew_flat = expert_weights.reshape(-1)
    counts = batch_size_per_expert.astype(jnp.int32)

    dispatch, combine = _sc_kernels()
    xbuf, inv3 = dispatch(x, ei_flat)
    ybuf = _gemm(counts, xbuf, W1, W2)
    out = combine(ybuf, inv3, ew_flat)
    return out


# speedup vs baseline: 1.0398x; 1.0398x over previous
import functools

import jax
import jax.numpy as jnp
from jax import lax
from jax.experimental import pallas as pl
from jax.experimental.pallas import tpu as pltpu
from jax.experimental.pallas import tpu_sc as plsc

N = 2048
D = 1024
F = 2048
E = 8
K = 2
NF = N * K
CAP = 1024
TM = 256
CB = CAP // TM
NC, NS, L = 2, 16, 16
NW = NC * NS
SPT = NF // NW
VPT = SPT // L
NCHUNK = 4
CH = SPT // NCHUNK


def _dispatch_body(x_hbm, ei_hbm, xbuf_hbm, inv_hbm,
                   ei_v, inv_idx, inv_even, inv_odd, shift, rows,
                   gsem, s0sem, s1sem):
    cid = lax.axis_index("c")
    sid = lax.axis_index("s")
    wid = sid * NC + cid
    base = wid * SPT

    g = pltpu.async_copy(x_hbm.at[pl.ds(wid * (SPT // K), SPT // K)], rows,
                         gsem)
    pltpu.sync_copy(ei_hbm, ei_v)
    shift[pl.ds(0, L)] = jnp.zeros((L,), jnp.int32)

    iota = lax.iota(jnp.int32, L)

    def prefix_incl(x):
        for k in (1, 2, 4, 8):
            shift[pl.ds(L, L)] = x
            x = x + shift[pl.ds(L - k, L)]
        return x

    def splat_last(x):
        return jnp.full((L,), x[L - 1], jnp.int32)

    def hist_step(j, accs):
        v = ei_v[pl.ds(j * L, L)]
        return tuple(
            acc + jnp.where(v == e, 1, 0) for e, acc in enumerate(accs))

    zeros = tuple(jnp.zeros((L,), jnp.int32) for _ in range(E))
    accs = lax.fori_loop(0, wid * VPT, hist_step, zeros)
    cnts = [splat_last(prefix_incl(a)) for a in accs]

    for j in range(VPT):
        v = ei_v[pl.ds(base + j * L, L)]
        inv_j = jnp.zeros((L,), jnp.int32)
        for e in range(E):
            m = v == e
            csum = prefix_incl(jnp.where(m, 1, 0))
            inv_j = jnp.where(m, e * CAP + cnts[e] + csum - 1, inv_j)
            cnts[e] = cnts[e] + splat_last(csum)
        c_idx = j // (VPT // NCHUNK)
        off = (j % (VPT // NCHUNK)) * L
        inv_idx[c_idx, pl.ds(off, L)] = inv_j

    def lane_gather(x, idx):
        return lax.gather(
            x, idx[:, None],
            dimension_numbers=lax.GatherDimensionNumbers(
                offset_dims=(), collapsed_slice_dims=(0,),
                start_index_map=(0,)),
            slice_sizes=(1,),
            mode=lax.GatherScatterMode.PROMISE_IN_BOUNDS)

    lo = jnp.minimum(2 * iota, L - 1)
    hi = jnp.maximum(2 * iota - L, 0)
    for p in range(VPT // 2):
        va = inv_idx[p, pl.ds(0, L)]
        vb = inv_idx[p, pl.ds(L, L)]
        sel = iota < (L // 2)
        ev = jnp.where(sel, lane_gather(va, lo), lane_gather(vb, hi))
        od = jnp.where(sel, lane_gather(va, jnp.minimum(lo + 1, L - 1)),
                       lane_gather(vb, jnp.minimum(hi + 1, L - 1)))
        inv_even[pl.ds(p * L, L)] = ev
        inv_odd[pl.ds(p * L, L)] = od

    pltpu.sync_copy(inv_idx, inv_hbm.at[wid])

    g.wait()
    s0 = pltpu.async_copy(rows, xbuf_hbm.at[inv_even], s0sem)
    s1 = pltpu.async_copy(rows, xbuf_hbm.at[inv_odd], s1sem)
    s0.wait()
    s1.wait()


def _gemm_body(counts_ref, x_ref, w1_ref, w2_ref, o_ref, w1b, w2b):
    e = pl.program_id(0)
    c = pl.program_id(1)

    @pl.when(c == 0)
    def _cast():
        w1b[...] = w1_ref[0].astype(jnp.bfloat16)
        w2b[...] = w2_ref[0].astype(jnp.bfloat16)

    @pl.when(c * TM < counts_ref[e])
    def _():
        x = x_ref[...].astype(jnp.bfloat16)
        h = jax.nn.relu(
            lax.dot_general(x, w1b[...], (((1,), (0,)), ((), ())),
                            preferred_element_type=jnp.float32)
        ).astype(jnp.bfloat16)
        o_ref[...] = lax.dot_general(h, w2b[...], (((1,), (0,)), ((), ())),
                                     preferred_element_type=jnp.float32)


def _combine_body(ybuf_hbm, inv_hbm, ew_hbm, out_hbm,
                  inv_idx, ew_v, rows0, rows1, orows, gsem):
    cid = lax.axis_index("c")
    sid = lax.axis_index("s")
    wid = sid * NC + cid

    pltpu.sync_copy(inv_hbm.at[wid], inv_idx)
    pltpu.sync_copy(ew_hbm.at[pl.ds(wid * SPT, SPT)], ew_v.at[pl.ds(0, SPT)])

    bufs = (rows0, rows1)
    g = pltpu.async_copy(ybuf_hbm.at[inv_idx.at[0]], bufs[0], gsem)
    for c in range(NCHUNK):
        g.wait()
        if c + 1 < NCHUNK:
            g_next = pltpu.async_copy(
                ybuf_hbm.at[inv_idx.at[c + 1]], bufs[(c + 1) % 2], gsem)
        rows = bufs[c % 2]

        def tbody(t, _):
            w0 = jnp.full((L,), ew_v[pl.ds(c * CH + 2 * t, L)][0], jnp.float32)
            w1 = jnp.full((L,), ew_v[pl.ds(c * CH + 2 * t + 1, L)][0],
                          jnp.float32)
            for jj in range(D // L):
                s = pl.ds(jj * L, L)
                orows[t, s] = w0 * rows[2 * t, s] + w1 * rows[2 * t + 1, s]
            return 0

        lax.fori_loop(0, CH // K, tbody, 0)
        pltpu.sync_copy(
            orows, out_hbm.at[pl.ds(wid * (SPT // K) + c * (CH // K), CH // K)])
        if c + 1 < NCHUNK:
            g = g_next


@functools.lru_cache(maxsize=None)
def _sc_kernels():
    mesh = plsc.VectorSubcoreMesh(
        core_axis_name="c", subcore_axis_name="s", num_cores=NC, num_subcores=NS)
    dispatch = pl.kernel(
        _dispatch_body,
        out_type=[
            jax.ShapeDtypeStruct((E * CAP, D), jnp.float32),
            jax.ShapeDtypeStruct((NW, NCHUNK, CH), jnp.int32),
        ],
        mesh=mesh,
        scratch_types=[
            pltpu.VMEM((NF,), jnp.int32),
            pltpu.VMEM((NCHUNK, CH), jnp.int32),
            pltpu.VMEM((SPT // K,), jnp.int32),
            pltpu.VMEM((SPT // K,), jnp.int32),
            pltpu.VMEM((2 * L,), jnp.int32),
            pltpu.VMEM((SPT // K, D), jnp.float32),
            pltpu.SemaphoreType.DMA,
            pltpu.SemaphoreType.DMA,
            pltpu.SemaphoreType.DMA,
        ],
    )
    combine = pl.kernel(
        _combine_body,
        out_type=jax.ShapeDtypeStruct((N, D), jnp.float32),
        mesh=mesh,
        scratch_types=[
            pltpu.VMEM((NCHUNK, CH), jnp.int32),
            pltpu.VMEM((SPT + L,), jnp.float32),
            pltpu.VMEM((CH, D), jnp.float32),
            pltpu.VMEM((CH, D), jnp.float32),
            pltpu.VMEM((CH // K, D), jnp.float32),
            pltpu.SemaphoreType.DMA,
        ],
    )
    return dispatch, combine

_gemm = pl.pallas_call(
    _gemm_body,
    grid_spec=pltpu.PrefetchScalarGridSpec(
        num_scalar_prefetch=1,
        grid=(E, CB),
        in_specs=[
            pl.BlockSpec(
                (TM, D),
                lambda e, c, cnt: (
                    e * CB + jnp.minimum(
                        c, jnp.maximum((cnt[e] + TM - 1) // TM - 1, 0)), 0)),
            pl.BlockSpec((1, D, F), lambda e, c, cnt: (e, 0, 0)),
            pl.BlockSpec((1, F, D), lambda e, c, cnt: (e, 0, 0)),
        ],
        out_specs=pl.BlockSpec(
            (TM, D),
            lambda e, c, cnt: (
                jnp.where(c * TM < cnt[e], e * CB + c, E * CB), 0)),
        scratch_shapes=[
            pltpu.VMEM((D, F), jnp.bfloat16),
            pltpu.VMEM((F, D), jnp.bfloat16),
        ],
    ),
    out_shape=jax.ShapeDtypeStruct(((E * CB + 1) * TM, D), jnp.float32),
)


def kernel(x, expert_weights, expert_indices, batch_size_per_expert, W1, W2):
    ei_flat = expert_indices.astype(jnp.int32).reshape(-1)
    ew_flat = expert_weights.reshape(-1)
    counts = batch_size_per_expert.astype(jnp.int32)

    dispatch, combine = _sc_kernels()
    xbuf, inv3 = dispatch(x, ei_flat)
    ybuf = _gemm(counts, xbuf, W1, W2)
    out = combine(ybuf, inv3, ew_flat)
    return out
